# baseline (speedup 1.0000x reference)
import functools

import jax
import jax.numpy as jnp
from jax import lax
from jax.experimental import pallas as pl
from jax.experimental.pallas import tpu as pltpu
from jax.experimental.pallas import tpu_sc as plsc

N = 10000
NP = 10240
DIN = 128
DH = 32
DOUT = 20
E = 320000
EPRIME = E + N
NEG_SLOPE = 0.2

NC = 2
NS = 16
CHUNK = 3200
EP = 332800
HALF = EP // NC

f32 = jnp.float32


def _make_sc_gat(F: int):
    FP1 = F + 1
    K = -(-F // NS)
    DTID = F % NS
    ACCN = K + 1
    mesh = plsc.VectorSubcoreMesh(
        core_axis_name="c", subcore_axis_name="s", num_cores=NC,
        num_subcores=NS)

    def body(src_hbm, dst_hbm, big_hbm, out_hbm, al_v, ar_v, *rest):
        xcols = rest[:K]
        src_v = rest[K:K + 2]
        dst_v = rest[K + 2:K + 4]
        in_sem, acc = rest[K + 4:]
        cid = lax.axis_index("c")
        tid = lax.axis_index("s")
        is_denom = tid == DTID
        dmask = jnp.broadcast_to(is_denom, (16,))

        pltpu.sync_copy(big_hbm.at[F + 1], al_v)
        pltpu.sync_copy(big_hbm.at[F + 2], ar_v)
        cols = [tid + NS * k for k in range(K)]
        colbase = [jnp.where(c < F, k * NP, K * NP)
                   for k, c in enumerate(cols)]
        for k in range(K):
            row = jnp.where(cols[k] < F, cols[k], F + 3)
            pltpu.sync_copy(big_hbm.at[row], xcols[k])
        for seg in range(ACCN):
            pltpu.sync_copy(big_hbm.at[F + 3], acc.at[pl.ds(seg * NP, NP)])

        nsteps = CHUNK // 16
        nchunks = HALF // CHUNK

        def issue_in(c, b):
            off = cid * HALF + c * CHUNK
            pltpu.async_copy(src_hbm.at[pl.ds(off, CHUNK)], src_v[b],
                             in_sem[b])
            pltpu.async_copy(dst_hbm.at[pl.ds(off, CHUNK)], dst_v[b],
                             in_sem[b])

        def wait_in(c, b):
            off = cid * HALF + c * CHUNK
            pltpu.make_async_copy(src_hbm.at[pl.ds(off, CHUNK)], src_v[b],
                                  in_sem[b]).wait()
            pltpu.make_async_copy(dst_hbm.at[pl.ds(off, CHUNK)], dst_v[b],
                                  in_sem[b]).wait()

        issue_in(0, 0)

        def pair(gp, carry):
            for b in range(2):
                c = 2 * gp + b

                @pl.when(c + 1 < nchunks)
                def _():
                    issue_in(c + 1, 1 - b)

                wait_in(c, b)

                def step(i, c2):
                    sl = pl.ds(i * 16, 16)
                    s = src_v[b][sl]
                    d = dst_v[b][sl]
                    a = plsc.load_gather(al_v, [s])
                    bb = plsc.load_gather(ar_v, [d])
                    e = a + bb
                    e = jnp.maximum(e, NEG_SLOPE * e)
                    w = jnp.exp(e)
                    for k in range(K):
                        xv = plsc.load_gather(xcols[k], [s])
                        plsc.addupdate_scatter(acc, [d + colbase[k]], xv * w)
                    plsc.addupdate_scatter(acc, [d + K * NP], w, mask=dmask)
                    return c2

                lax.fori_loop(0, nsteps, step, 0, unroll=8)

            return carry

        lax.fori_loop(0, nchunks // 2, pair, 0)

        for k in range(K):
            @pl.when(cols[k] < F)
            def _():
                pltpu.sync_copy(acc.at[pl.ds(k * NP, NP)],
                                out_hbm.at[cid, cols[k]])

        @pl.when(tid == DTID)
        def _():
            fcol = tid + (F - DTID)
            pltpu.sync_copy(acc.at[pl.ds(K * NP, NP)],
                            out_hbm.at[cid, fcol])

    return pl.kernel(
        body,
        out_type=jax.ShapeDtypeStruct((NC, FP1, NP), f32),
        mesh=mesh,
        compiler_params=pltpu.CompilerParams(needs_layout_passes=False),
        scratch_types=[
            pltpu.VMEM((NP,), f32),
            pltpu.VMEM((NP,), f32),
            *[pltpu.VMEM((NP,), f32) for _ in range(K)],
            *[pltpu.VMEM((CHUNK,), jnp.int32) for _ in range(2)],
            *[pltpu.VMEM((CHUNK,), jnp.int32) for _ in range(2)],
            [pltpu.SemaphoreType.DMA, pltpu.SemaphoreType.DMA],
            pltpu.VMEM((ACCN * NP,), f32),
        ],
    )


_sc_gat1 = _make_sc_gat(DH)
_sc_gat2 = _make_sc_gat(DOUT)


def _tc1_body(xp_ref, w0_ref, b0_ref, wl_ref, wr_ref, atl_ref, atr_ref,
              big_ref, sm0_ref):
    xp = xp_ref[...]
    dn = (((1,), (1,)), ((), ()))
    xlT = lax.dot_general(wl_ref[...], xp, dn, preferred_element_type=f32)
    xrT = lax.dot_general(wr_ref[...], xp, dn, preferred_element_type=f32)
    dn0 = (((1,), (0,)), ((), ()))
    al = lax.dot_general(atl_ref[...], xlT, dn0, preferred_element_type=f32)
    ar = lax.dot_general(atr_ref[...], xrT, dn0, preferred_element_type=f32)
    ones = jnp.ones((1, NP), f32)
    zeros = jnp.zeros((1, NP), f32)
    big_ref[...] = jnp.concatenate([xlT, ones, al, ar, zeros], axis=0)

    s0T = lax.dot_general(w0_ref[...], xp, dn, preferred_element_type=f32)
    s0T = s0T + b0_ref[...]
    colid = lax.broadcasted_iota(jnp.int32, (DOUT, NP), 1)
    s0T = jnp.where(colid < N, s0T, -jnp.inf)
    m = jnp.max(s0T, axis=1, keepdims=True)
    ex = jnp.exp(s0T - m)
    sm0_ref[...] = ex / jnp.sum(ex, axis=1, keepdims=True)


_tc1 = pl.pallas_call(
    _tc1_body,
    out_shape=(jax.ShapeDtypeStruct((DH + 4, NP), f32),
               jax.ShapeDtypeStruct((DOUT, NP), f32)),
)


def _tc2_body(acc_ref, b1_ref, wl_ref, wr_ref, atl_ref, atr_ref, big_ref):
    a = acc_ref[0] + acc_ref[1]
    h = a[0:DH, :] / (a[DH:DH + 1, :] + 1e-16) + b1_ref[...]
    h = jnp.maximum(h, 0.0)
    dn = (((1,), (0,)), ((), ()))
    xlT = lax.dot_general(wl_ref[...], h, dn, preferred_element_type=f32)
    xrT = lax.dot_general(wr_ref[...], h, dn, preferred_element_type=f32)
    al = lax.dot_general(atl_ref[...], xlT, dn, preferred_element_type=f32)
    ar = lax.dot_general(atr_ref[...], xrT, dn, preferred_element_type=f32)
    ones = jnp.ones((1, NP), f32)
    zeros = jnp.zeros((1, NP), f32)
    big_ref[...] = jnp.concatenate([xlT, ones, al, ar, zeros], axis=0)


_tc2 = pl.pallas_call(
    _tc2_body,
    out_shape=jax.ShapeDtypeStruct((DOUT + 4, NP), f32),
)


def _tc3_body(acc_ref, b2_ref, sm0_ref, out_ref):
    a = acc_ref[0] + acc_ref[1]
    g = a[0:DOUT, :] / (a[DOUT:DOUT + 1, :] + 1e-16) + b2_ref[...]
    m = jnp.max(g, axis=0, keepdims=True)
    ex = jnp.exp(g - m)
    s1 = ex / jnp.sum(ex, axis=0, keepdims=True)
    out_ref[...] = sm0_ref[...] * s1


_tc3 = pl.pallas_call(
    _tc3_body,
    out_shape=jax.ShapeDtypeStruct((DOUT, NP), f32),
)


def kernel(x, edges, W0, b0, Wl1, Wr1, attl1, attr1, bias1,
           Wl2, Wr2, attl2, attr2, bias2):
    xp = jnp.pad(x, ((0, NP - N), (0, 0)))
    loop = jnp.arange(N, dtype=jnp.int32)
    padi = jnp.full((EP - EPRIME,), N, jnp.int32)
    src = jnp.concatenate([edges[0], loop, padi])
    dst = jnp.concatenate([edges[1], loop, padi])

    big1, sm0T = _tc1(xp, W0, b0.reshape(DOUT, 1), Wl1, Wr1,
                      attl1.reshape(1, DH), attr1.reshape(1, DH))
    acc1 = _sc_gat1(src, dst, big1)
    big2 = _tc2(acc1, bias1.reshape(DH, 1), Wl2, Wr2,
                attl2.reshape(1, DOUT), attr2.reshape(1, DOUT))
    acc2 = _sc_gat2(src, dst, big2)
    milT = _tc3(acc2, bias2.reshape(DOUT, 1), sm0T)
    return milT[:, :N].T


# speedup vs baseline: 1.2810x; 1.2810x over previous
import functools

import jax
import jax.numpy as jnp
from jax import lax
from jax.experimental import pallas as pl
from jax.experimental.pallas import tpu as pltpu
from jax.experimental.pallas import tpu_sc as plsc

N = 10000
NP = 10240
DIN = 128
DH = 32
DOUT = 20
E = 320000
EPRIME = E + N
NEG_SLOPE = 0.2

NC = 2
NS = 16
CHUNK = 3200
EP = 332800
HALF = EP // NC

f32 = jnp.float32


def _make_sc_gat(F: int):
    FP1 = F + 1
    K = -(-F // NS)
    DTID = F % NS
    ACCN = K + 1
    mesh = plsc.VectorSubcoreMesh(
        core_axis_name="c", subcore_axis_name="s", num_cores=NC,
        num_subcores=NS)

    def body(src_hbm, dst_hbm, big_hbm, out_hbm, ea1, er1, ea2, er2, *rest):
        xcols = rest[:K]
        src_v = rest[K:K + 2]
        dst_v = rest[K + 2:K + 4]
        in_sem, acc = rest[K + 4:]
        cid = lax.axis_index("c")
        tid = lax.axis_index("s")
        is_denom = tid == DTID
        dmask = jnp.broadcast_to(is_denom, (16,))

        pltpu.sync_copy(big_hbm.at[F + 0], ea1)
        pltpu.sync_copy(big_hbm.at[F + 1], er1)
        pltpu.sync_copy(big_hbm.at[F + 2], ea2)
        pltpu.sync_copy(big_hbm.at[F + 3], er2)
        cols = [tid + NS * k for k in range(K)]
        colbase = [jnp.where(c < F, k * NP, K * NP)
                   for k, c in enumerate(cols)]
        for k in range(K):
            row = jnp.where(cols[k] < F, cols[k], F + 4)
            pltpu.sync_copy(big_hbm.at[row], xcols[k])
        for seg in range(ACCN):
            pltpu.sync_copy(big_hbm.at[F + 4], acc.at[pl.ds(seg * NP, NP)])

        nsteps = CHUNK // 16
        nchunks = HALF // CHUNK

        def issue_in(c, b):
            off = cid * HALF + c * CHUNK
            pltpu.async_copy(src_hbm.at[pl.ds(off, CHUNK)], src_v[b],
                             in_sem[b])
            pltpu.async_copy(dst_hbm.at[pl.ds(off, CHUNK)], dst_v[b],
                             in_sem[b])

        def wait_in(c, b):
            off = cid * HALF + c * CHUNK
            pltpu.make_async_copy(src_hbm.at[pl.ds(off, CHUNK)], src_v[b],
                                  in_sem[b]).wait()
            pltpu.make_async_copy(dst_hbm.at[pl.ds(off, CHUNK)], dst_v[b],
                                  in_sem[b]).wait()

        issue_in(0, 0)

        def pair(gp, carry):
            for b in range(2):
                c = 2 * gp + b

                @pl.when(c + 1 < nchunks)
                def _():
                    issue_in(c + 1, 1 - b)

                wait_in(c, b)

                def step(i, c2):
                    sl = pl.ds(i * 16, 16)
                    s = src_v[b][sl]
                    d = dst_v[b][sl]
                    p = plsc.load_gather(ea1, [s])
                    q = plsc.load_gather(er1, [d])
                    r = plsc.load_gather(ea2, [s])
                    t = plsc.load_gather(er2, [d])
                    e1 = p * q
                    w = jnp.where(e1 > 1.0, e1, r * t)
                    for k in range(K):
                        xv = plsc.load_gather(xcols[k], [s])
                        plsc.addupdate_scatter(acc, [d + colbase[k]], xv * w)
                    plsc.addupdate_scatter(acc, [d + K * NP], w, mask=dmask)
                    return c2

                lax.fori_loop(0, nsteps, step, 0, unroll=8)

            return carry

        lax.fori_loop(0, nchunks // 2, pair, 0)

        for k in range(K):
            @pl.when(cols[k] < F)
            def _():
                pltpu.sync_copy(acc.at[pl.ds(k * NP, NP)],
                                out_hbm.at[cid, cols[k]])

        @pl.when(tid == DTID)
        def _():
            fcol = tid + (F - DTID)
            pltpu.sync_copy(acc.at[pl.ds(K * NP, NP)],
                            out_hbm.at[cid, fcol])

    return pl.kernel(
        body,
        out_type=jax.ShapeDtypeStruct((NC, FP1, NP), f32),
        mesh=mesh,
        compiler_params=pltpu.CompilerParams(needs_layout_passes=False),
        scratch_types=[
            pltpu.VMEM((NP,), f32),
            pltpu.VMEM((NP,), f32),
            pltpu.VMEM((NP,), f32),
            pltpu.VMEM((NP,), f32),
            *[pltpu.VMEM((NP,), f32) for _ in range(K)],
            *[pltpu.VMEM((CHUNK,), jnp.int32) for _ in range(2)],
            *[pltpu.VMEM((CHUNK,), jnp.int32) for _ in range(2)],
            [pltpu.SemaphoreType.DMA, pltpu.SemaphoreType.DMA],
            pltpu.VMEM((ACCN * NP,), f32),
        ],
    )


_sc_gat1 = _make_sc_gat(DH)
_sc_gat2 = _make_sc_gat(DOUT)


def _tc1_body(xp_ref, w0_ref, b0_ref, wl_ref, wr_ref, atl_ref, atr_ref,
              big_ref, sm0_ref):
    xp = xp_ref[...]
    dn = (((1,), (1,)), ((), ()))
    xlT = lax.dot_general(wl_ref[...], xp, dn, preferred_element_type=f32)
    xrT = lax.dot_general(wr_ref[...], xp, dn, preferred_element_type=f32)
    dn0 = (((1,), (0,)), ((), ()))
    al = lax.dot_general(atl_ref[...], xlT, dn0, preferred_element_type=f32)
    ar = lax.dot_general(atr_ref[...], xrT, dn0, preferred_element_type=f32)
    zeros = jnp.zeros((-(-(DH + 5) // 8) * 8 - DH - 4, NP), f32)
    big_ref[...] = jnp.concatenate(
        [xlT, jnp.exp(al), jnp.exp(ar),
         jnp.exp(NEG_SLOPE * al), jnp.exp(NEG_SLOPE * ar), zeros], axis=0)

    s0T = lax.dot_general(w0_ref[...], xp, dn, preferred_element_type=f32)
    s0T = s0T + b0_ref[...]
    colid = lax.broadcasted_iota(jnp.int32, (DOUT, NP), 1)
    s0T = jnp.where(colid < N, s0T, -jnp.inf)
    m = jnp.max(s0T, axis=1, keepdims=True)
    ex = jnp.exp(s0T - m)
    sm0_ref[...] = ex / jnp.sum(ex, axis=1, keepdims=True)


_tc1 = pl.pallas_call(
    _tc1_body,
    out_shape=(jax.ShapeDtypeStruct((-(-(DH + 5) // 8) * 8, NP), f32),
               jax.ShapeDtypeStruct((DOUT, NP), f32)),
)


def _tc2_body(acc_ref, b1_ref, wl_ref, wr_ref, atl_ref, atr_ref, big_ref):
    a = acc_ref[0] + acc_ref[1]
    h = a[0:DH, :] / (a[DH:DH + 1, :] + 1e-16) + b1_ref[...]
    h = jnp.maximum(h, 0.0)
    dn = (((1,), (0,)), ((), ()))
    xlT = lax.dot_general(wl_ref[...], h, dn, preferred_element_type=f32)
    xrT = lax.dot_general(wr_ref[...], h, dn, preferred_element_type=f32)
    al = lax.dot_general(atl_ref[...], xlT, dn, preferred_element_type=f32)
    ar = lax.dot_general(atr_ref[...], xrT, dn, preferred_element_type=f32)
    zeros = jnp.zeros((-(-(DOUT + 5) // 8) * 8 - DOUT - 4, NP), f32)
    big_ref[...] = jnp.concatenate(
        [xlT, jnp.exp(al), jnp.exp(ar),
         jnp.exp(NEG_SLOPE * al), jnp.exp(NEG_SLOPE * ar), zeros], axis=0)


_tc2 = pl.pallas_call(
    _tc2_body,
    out_shape=jax.ShapeDtypeStruct((-(-(DOUT + 5) // 8) * 8, NP), f32),
)


def _tc3_body(acc_ref, b2_ref, sm0_ref, out_ref):
    a = acc_ref[0] + acc_ref[1]
    g = a[0:DOUT, :] / (a[DOUT:DOUT + 1, :] + 1e-16) + b2_ref[...]
    m = jnp.max(g, axis=0, keepdims=True)
    ex = jnp.exp(g - m)
    s1 = ex / jnp.sum(ex, axis=0, keepdims=True)
    out_ref[...] = sm0_ref[...] * s1


_tc3 = pl.pallas_call(
    _tc3_body,
    out_shape=jax.ShapeDtypeStruct((DOUT, NP), f32),
)


def kernel(x, edges, W0, b0, Wl1, Wr1, attl1, attr1, bias1,
           Wl2, Wr2, attl2, attr2, bias2):
    xp = jnp.pad(x, ((0, NP - N), (0, 0)))
    loop = jnp.arange(N, dtype=jnp.int32)
    padi = jnp.full((EP - EPRIME,), N, jnp.int32)
    src = jnp.concatenate([edges[0], loop, padi])
    dst = jnp.concatenate([edges[1], loop, padi])

    big1, sm0T = _tc1(xp, W0, b0.reshape(DOUT, 1), Wl1, Wr1,
                      attl1.reshape(1, DH), attr1.reshape(1, DH))
    acc1 = _sc_gat1(src, dst, big1)
    big2 = _tc2(acc1, bias1.reshape(DH, 1), Wl2, Wr2,
                attl2.reshape(1, DOUT), attr2.reshape(1, DOUT))
    acc2 = _sc_gat2(src, dst, big2)
    milT = _tc3(acc2, bias2.reshape(DOUT, 1), sm0T)
    return milT[:, :N].T
